# baseline (speedup 1.0000x reference)
import functools

import numpy as np
import jax
import jax.numpy as jnp
from jax import lax
from jax.experimental import pallas as pl
from jax.experimental.pallas import tpu as pltpu
from jax.experimental.pallas import tpu_sc as plsc

_NA = 80
_KS = 11
_KP = 101
_GD_S = np.float32(30.0 / (_KS - 1))
_GD_P = np.float32(10.0 / (_KP - 1))
_T_HI_P = np.float32(99.0)
_T_HI_S = np.float32(9.0)

_NCH = 320
_NPIX = 128 * 128
_NW = 32
_CPW = _NCH // _NW


def _project_phi(src, tmp, dst, ks_i):
    f32 = jnp.float32
    tmp[pl.ds(0, 16)] = jnp.zeros((16,), f32)
    carry = np.float32(0.0)
    for v in range(8):
        k0 = 16 * v
        a = src[pl.ds(k0, 16)]
        b = src[pl.ds(k0 + 1, 16)]
        sl = jnp.clip((b - a) / _GD_P, 0.0, 1.0)
        kv = ks_i + k0
        sl = jnp.where((kv >= 1) & (kv <= _KP - 3), sl, np.float32(0.0))
        inc = plsc.cumsum(sl) + carry
        tmp[pl.ds(16 + k0, 16)] = inc
        carry = inc[15]
    for v in range(8):
        k0 = 16 * v
        csk = tmp[pl.ds(15 + k0, 16)]
        ridx = jnp.maximum((_KP - 2 + 16) - (ks_i + k0), 0)
        csr = plsc.load_gather(tmp, [ridx])
        dst[pl.ds(k0, 16)] = np.float32(0.5 * 0.1) * (csk - csr)


def _sc_body(x_hbm, sig_hbm, ac_hbm, mc_hbm, pp_hbm, pm_hbm, out_hbm,
             tbl_a, tbl_d, xbuf, obuf,
             sig_s, ac_s, mc_s, pp_s, pm_s, tmp_s, app_s, apm_s,
             sem_in, sem_out, sem_tbl):
    f32 = jnp.float32
    cid = lax.axis_index("c")
    sid = lax.axis_index("s")
    wid = sid * 2 + cid
    ch0 = wid * _CPW
    e0 = ch0 * _NPIX

    in_copies = {0: pltpu.async_copy(x_hbm.at[pl.ds(e0, _NPIX)],
                                     xbuf.at[0], sem_in)}

    small_copies = [
        pltpu.async_copy(sig_hbm, sig_s, sem_tbl),
        pltpu.async_copy(ac_hbm, ac_s, sem_tbl),
        pltpu.async_copy(mc_hbm, mc_s.at[pl.ds(0, _KS)], sem_tbl),
        pltpu.async_copy(pp_hbm, pp_s.at[pl.ds(0, _KP)], sem_tbl),
        pltpu.async_copy(pm_hbm, pm_s.at[pl.ds(0, _KP)], sem_tbl),
    ]
    for c in small_copies:
        c.wait()

    ks_i = lax.iota(jnp.int32, 16)
    ks_f = ks_i.astype(f32)
    _project_phi(pp_s, tmp_s, app_s, ks_i)
    _project_phi(pm_s, tmp_s, apm_s, ks_i)

    c_idx = ch0 + jnp.minimum(ks_i, _CPW - 1)
    sigv = plsc.load_gather(sig_s, [c_idx])
    t = sigv / _GD_S
    t_cl = jnp.clip(t, 0.0, _T_HI_S)
    i_s = t_cl.astype(jnp.int32)
    fr = t - i_s.astype(f32)
    bidx = ((c_idx >= _NA).astype(jnp.int32)
            + (c_idx >= 2 * _NA).astype(jnp.int32)
            + (c_idx >= 3 * _NA).astype(jnp.int32))
    cm = c_idx - _NA * bidx
    ai = cm * _KS + i_s
    a0 = plsc.load_gather(ac_s, [ai])
    a1 = plsc.load_gather(ac_s, [ai + 1])
    alpha_s = a1 * fr + a0 * (np.float32(1.0) - fr)
    m0 = plsc.load_gather(mc_s, [i_s])
    m1 = plsc.load_gather(mc_s, [i_s + 1])
    mu = m1 * fr + m0 * (np.float32(1.0) - fr)
    r = (sigv + np.float32(1e-5)) / jnp.exp(alpha_s)
    s2v = r * r
    s1v = mu * s2v

    for ch in range(_CPW):
        s1c = s1v[ch]
        s2c = s2v[ch]
        for v in range(8):
            k0 = 16 * v
            av = s1c * app_s[pl.ds(k0, 16)] - s2c * apm_s[pl.ds(k0, 16)]
            an = s1c * app_s[pl.ds(k0 + 1, 16)] - s2c * apm_s[pl.ds(k0 + 1, 16)]
            dv = an - av
            kb = ks_f + np.float32(k0 + ch * 128)
            tbl_a[pl.ds(ch * 128 + k0, 16)] = av - kb * dv
            tbl_d[pl.ds(ch * 128 + k0, 16)] = dv

    out_copies = {}
    for ch in range(_CPW):
        b = ch % 2
        if ch + 1 < _CPW:
            in_copies[ch + 1] = pltpu.async_copy(
                x_hbm.at[pl.ds(e0 + (ch + 1) * _NPIX, _NPIX)],
                xbuf.at[1 - b], sem_in)
        in_copies[ch].wait()
        if ch >= 2:
            out_copies[ch - 2].wait()

        c0 = np.float32(50.0 + ch * 128)
        lo = np.float32(ch * 128)
        hi = np.float32(ch * 128) + _T_HI_P

        @plsc.parallel_loop(0, _NPIX, step=16, unroll=8)
        def _inner(base, _b=b, _c0=c0, _lo=lo, _hi=hi):
            xv = xbuf[_b, pl.ds(base, 16)]
            t = xv * np.float32(10.0) + _c0
            idx = jnp.clip(t, _lo, _hi).astype(jnp.int32)
            av = plsc.load_gather(tbl_a, [idx])
            dv = plsc.load_gather(tbl_d, [idx])
            obuf[_b, pl.ds(base, 16)] = av + t * dv

        out_copies[ch] = pltpu.async_copy(
            obuf.at[b], out_hbm.at[pl.ds(e0 + ch * _NPIX, _NPIX)], sem_out)
    out_copies[_CPW - 2].wait()
    out_copies[_CPW - 1].wait()


@functools.cache
def _sc_call():
    f32 = jnp.float32
    return pl.kernel(
        _sc_body,
        out_type=jax.ShapeDtypeStruct((_NCH * _NPIX,), f32),
        mesh=plsc.VectorSubcoreMesh(core_axis_name="c", subcore_axis_name="s",
                                    num_cores=2, num_subcores=16),
        scratch_types=[
            pltpu.VMEM((_CPW * 128,), f32),
            pltpu.VMEM((_CPW * 128,), f32),
            pltpu.VMEM((2, _NPIX), f32),
            pltpu.VMEM((2, _NPIX), f32),
            pltpu.VMEM((_NCH,), f32),
            pltpu.VMEM((_NA * _KS,), f32),
            pltpu.VMEM((16,), f32),
            pltpu.VMEM((144,), f32),
            pltpu.VMEM((144,), f32),
            pltpu.VMEM((160,), f32),
            pltpu.VMEM((160,), f32),
            pltpu.VMEM((160,), f32),
            pltpu.SemaphoreType.DMA,
            pltpu.SemaphoreType.DMA,
            pltpu.SemaphoreType.DMA,
        ],
        compiler_params=pltpu.CompilerParams(needs_layout_passes=False),
    )


def kernel(x, sigma, alpha_coeffs, mu_coeffs, phi_plus_coeffs, phi_minus_coeffs):
    out = _sc_call()(x.reshape(-1), sigma.reshape(-1),
                     alpha_coeffs.reshape(-1), mu_coeffs.reshape(-1),
                     phi_plus_coeffs.reshape(-1), phi_minus_coeffs.reshape(-1))
    return out.reshape(x.shape)


# speedup vs baseline: 1.0022x; 1.0022x over previous
import functools

import numpy as np
import jax
import jax.numpy as jnp
from jax import lax
from jax.experimental import pallas as pl
from jax.experimental.pallas import tpu as pltpu
from jax.experimental.pallas import tpu_sc as plsc

_NA = 80
_KS = 11
_KP = 101
_GD_S = np.float32(30.0 / (_KS - 1))
_GD_P = np.float32(10.0 / (_KP - 1))
_T_HI_P = np.float32(99.0)
_T_HI_S = np.float32(9.0)

_NCH = 320
_NPIX = 128 * 128
_NW = 32
_CPW = _NCH // _NW

_SIG0 = 0
_AC0 = _SIG0 + _NCH
_MC0 = _AC0 + _NA * _KS
_PP0 = _MC0 + _KS
_PM0 = _PP0 + _KP
_NSM = _PM0 + _KP


def _project_phi(src, off, tmp, dst, ks_i):
    f32 = jnp.float32
    tmp[pl.ds(0, 16)] = jnp.zeros((16,), f32)
    carry = np.float32(0.0)
    for v in range(8):
        k0 = 16 * v
        a = src[pl.ds(off + k0, 16)]
        b = src[pl.ds(off + k0 + 1, 16)]
        sl = jnp.clip((b - a) / _GD_P, 0.0, 1.0)
        kv = ks_i + k0
        sl = jnp.where((kv >= 1) & (kv <= _KP - 3), sl, np.float32(0.0))
        inc = plsc.cumsum(sl) + carry
        tmp[pl.ds(16 + k0, 16)] = inc
        carry = inc[15]
    for v in range(8):
        k0 = 16 * v
        csk = tmp[pl.ds(15 + k0, 16)]
        ridx = jnp.maximum((_KP - 2 + 16) - (ks_i + k0), 0)
        csr = plsc.load_gather(tmp, [ridx])
        dst[pl.ds(k0, 16)] = np.float32(0.5 * 0.1) * (csk - csr)


def _sc_body(x_hbm, sm_hbm, out_hbm,
             tbl_a, tbl_d, xbuf, obuf,
             sm_s, tmp_s, app_s, apm_s,
             sem_in, sem_out, sem_tbl):
    f32 = jnp.float32
    cid = lax.axis_index("c")
    sid = lax.axis_index("s")
    wid = sid * 2 + cid
    ch0 = wid * _CPW
    e0 = ch0 * _NPIX

    in_copies = {0: pltpu.async_copy(x_hbm.at[pl.ds(e0, _NPIX)],
                                     xbuf.at[0], sem_in)}

    tbl_copy = pltpu.async_copy(sm_hbm, sm_s.at[pl.ds(0, _NSM)], sem_tbl)
    tbl_copy.wait()

    ks_i = lax.iota(jnp.int32, 16)
    ks_f = ks_i.astype(f32)
    _project_phi(sm_s, _PP0, tmp_s, app_s, ks_i)
    _project_phi(sm_s, _PM0, tmp_s, apm_s, ks_i)

    c_idx = ch0 + jnp.minimum(ks_i, _CPW - 1)
    sigv = plsc.load_gather(sm_s, [c_idx])
    t = sigv / _GD_S
    t_cl = jnp.clip(t, 0.0, _T_HI_S)
    i_s = t_cl.astype(jnp.int32)
    fr = t - i_s.astype(f32)
    bidx = ((c_idx >= _NA).astype(jnp.int32)
            + (c_idx >= 2 * _NA).astype(jnp.int32)
            + (c_idx >= 3 * _NA).astype(jnp.int32))
    cm = c_idx - _NA * bidx
    ai = cm * _KS + (i_s + _AC0)
    a0 = plsc.load_gather(sm_s, [ai])
    a1 = plsc.load_gather(sm_s, [ai + 1])
    alpha_s = a1 * fr + a0 * (np.float32(1.0) - fr)
    m0 = plsc.load_gather(sm_s, [i_s + _MC0])
    m1 = plsc.load_gather(sm_s, [i_s + _MC0 + 1])
    mu = m1 * fr + m0 * (np.float32(1.0) - fr)
    r = (sigv + np.float32(1e-5)) / jnp.exp(alpha_s)
    s2v = r * r
    s1v = mu * s2v

    for ch in range(_CPW):
        s1c = s1v[ch]
        s2c = s2v[ch]
        for v in range(8):
            k0 = 16 * v
            av = s1c * app_s[pl.ds(k0, 16)] - s2c * apm_s[pl.ds(k0, 16)]
            an = s1c * app_s[pl.ds(k0 + 1, 16)] - s2c * apm_s[pl.ds(k0 + 1, 16)]
            dv = an - av
            kb = ks_f + np.float32(k0 + ch * 128)
            tbl_a[pl.ds(ch * 128 + k0, 16)] = av - kb * dv
            tbl_d[pl.ds(ch * 128 + k0, 16)] = dv

    out_copies = {}
    for ch in range(_CPW):
        b = ch % 2
        if ch + 1 < _CPW:
            in_copies[ch + 1] = pltpu.async_copy(
                x_hbm.at[pl.ds(e0 + (ch + 1) * _NPIX, _NPIX)],
                xbuf.at[1 - b], sem_in)
        in_copies[ch].wait()
        if ch >= 2:
            out_copies[ch - 2].wait()

        c0 = np.float32(50.0 + ch * 128)
        lo = np.float32(ch * 128)
        hi = np.float32(ch * 128) + _T_HI_P

        @plsc.parallel_loop(0, _NPIX, step=16, unroll=8)
        def _inner(base, _b=b, _c0=c0, _lo=lo, _hi=hi):
            xv = xbuf[_b, pl.ds(base, 16)]
            t = xv * np.float32(10.0) + _c0
            idx = jnp.clip(t, _lo, _hi).astype(jnp.int32)
            av = plsc.load_gather(tbl_a, [idx])
            dv = plsc.load_gather(tbl_d, [idx])
            obuf[_b, pl.ds(base, 16)] = av + t * dv

        out_copies[ch] = pltpu.async_copy(
            obuf.at[b], out_hbm.at[pl.ds(e0 + ch * _NPIX, _NPIX)], sem_out)
    out_copies[_CPW - 2].wait()
    out_copies[_CPW - 1].wait()


@functools.cache
def _sc_call():
    f32 = jnp.float32
    return pl.kernel(
        _sc_body,
        out_type=jax.ShapeDtypeStruct((_NCH * _NPIX,), f32),
        mesh=plsc.VectorSubcoreMesh(core_axis_name="c", subcore_axis_name="s",
                                    num_cores=2, num_subcores=16),
        scratch_types=[
            pltpu.VMEM((_CPW * 128,), f32),
            pltpu.VMEM((_CPW * 128,), f32),
            pltpu.VMEM((2, _NPIX), f32),
            pltpu.VMEM((2, _NPIX), f32),
            pltpu.VMEM((_NSM + 43,), f32),
            pltpu.VMEM((160,), f32),
            pltpu.VMEM((160,), f32),
            pltpu.VMEM((160,), f32),
            pltpu.SemaphoreType.DMA,
            pltpu.SemaphoreType.DMA,
            pltpu.SemaphoreType.DMA,
        ],
        compiler_params=pltpu.CompilerParams(needs_layout_passes=False),
    )


def kernel(x, sigma, alpha_coeffs, mu_coeffs, phi_plus_coeffs, phi_minus_coeffs):
    smalls = jnp.concatenate(
        [sigma.reshape(-1), alpha_coeffs.reshape(-1), mu_coeffs.reshape(-1),
         phi_plus_coeffs.reshape(-1), phi_minus_coeffs.reshape(-1)])
    out = _sc_call()(x.reshape(-1), smalls)
    return out.reshape(x.shape)
